# baseline (speedup 1.0000x reference)
import jax
import jax.numpy as jnp
from jax import lax
from jax.experimental import pallas as pl
from jax.experimental.pallas import tpu as pltpu
from jax.experimental.pallas import tpu_sc as plsc

B = 256
HW = 28
N = 38400
SX = 32
IMG = 30 * SX
RTOT = B * IMG
NW = 32
IPT = B // NW
RCH = IPT * IMG
PCH = 1280
NB = 8
RS = NB * IMG


def _sc_body(flat_hbm, feat_hbm, p1_hbm, m2_hbm, flat_c, feat_c, p1_v, m2_v,
             srt_v):
    wid = lax.axis_index("s") * 2 + lax.axis_index("c")
    rbase = wid * RCH
    lanes = lax.iota(jnp.int32, 16)
    zf = jnp.zeros((16,), jnp.float32)
    onef = jnp.ones((16,), jnp.float32)

    def zero_body(v, c):
        rows = v * 16 + lanes
        plsc.store_scatter(p1_v, [rows * 16 + 4], zf)
        plsc.store_scatter(p1_v, [rows * 16 + 15], zf)
        return c

    lax.fori_loop(0, RCH // 16, zero_body, 0)

    for j in range(4):
        srt_v[pl.ds(j * 32 + 16, 16)] = jnp.full((16,), -1, jnp.int32)
    big = RCH * 16

    def chunk_body(ci, c):
        pltpu.sync_copy(flat_hbm.at[pl.ds(ci * PCH, PCH)], flat_c)
        pltpu.sync_copy(feat_hbm.at[pl.ds(ci * PCH, PCH)], feat_c)

        def body(i, c2):
            for j in range(4):
                v = i * 4 + j
                rel = flat_c[pl.ds(v * 16, 16)] - rbase
                m = (rel >= 0) & (rel < RCH)
                relc = jnp.where(m, rel, 0)
                f = feat_c[pl.ds(v * 16, 16)]
                key = jnp.where(m, rel * 16 + lanes, big + lanes)
                sk, sv = plsc.sort_key_val(key, f)
                srt_v[pl.ds(j * 32, 16)] = sk
                nxt = srt_v[pl.ds(j * 32 + 1, 16)]
                win = ((sk >> 4) != (nxt >> 4)) & (sk < big)
                dst = jnp.where(sk < big, (sk & -16) + 4, 0)
                plsc.store_scatter(p1_v, [dst], sv, mask=win)
                plsc.store_scatter(p1_v, [relc * 16 + 15], onef, mask=m)
            return c2

        lax.fori_loop(0, PCH // 64, body, 0)
        return c

    lax.fori_loop(0, N // PCH, chunk_body, 0)

    def exp_body(v, c):
        rows = v * 16 + lanes
        for k in range(9):
            if k == 4:
                continue
            d = (k // 3 - 1) * SX + (k % 3 - 1)
            src = jnp.clip(rows + d, 0, RCH - 1) * 16 + 4
            vals = plsc.load_gather(p1_v, [src])
            plsc.store_scatter(p1_v, [rows * 16 + k], vals)
        return c

    lax.fori_loop(0, RCH // 16, exp_body, 0)

    def m2_body(v, c):
        l = v * 16 + lanes
        li = l // 196
        pos = l % 196
        y = pos // 14
        x = pos % 14
        r2 = li * IMG + (2 * y + 1) * SX + (2 * x + 1)
        g0 = plsc.load_gather(p1_v, [r2 * 16 + 15])
        g1 = plsc.load_gather(p1_v, [(r2 + 1) * 16 + 15])
        g2 = plsc.load_gather(p1_v, [(r2 + SX) * 16 + 15])
        g3 = plsc.load_gather(p1_v, [(r2 + SX + 1) * 16 + 15])
        m2_v[pl.ds(v * 16, 16)] = jnp.maximum(jnp.maximum(g0, g1),
                                              jnp.maximum(g2, g3))
        return c

    lax.fori_loop(0, IPT * 196 // 16, m2_body, 0)

    pltpu.sync_copy(p1_v, p1_hbm.at[pl.ds(rbase * 16, RCH * 16)])
    pltpu.sync_copy(m2_v, m2_hbm.at[pl.ds(wid * IPT * 196, IPT * 196)])


def _densify(flat, feat):
    mesh = plsc.VectorSubcoreMesh(core_axis_name="c", subcore_axis_name="s")
    return pl.kernel(
        _sc_body,
        mesh=mesh,
        out_type=[jax.ShapeDtypeStruct((RTOT * 16,), jnp.float32),
                  jax.ShapeDtypeStruct((B * 196,), jnp.float32)],
        scratch_types=[pltpu.VMEM((PCH,), jnp.int32),
                       pltpu.VMEM((PCH,), jnp.float32),
                       pltpu.VMEM((RCH * 16,), jnp.float32),
                       pltpu.VMEM((IPT * 196,), jnp.float32),
                       pltpu.VMEM((128,), jnp.int32)],
        compiler_params=pltpu.CompilerParams(needs_layout_passes=False),
    )(flat, feat)


def _conv_body(p1_ref, m2_ref, w1r, wmr, bi1r, w2r, g2r, b2r,
               m2r_, v2r, w3r, g3r, b3r, m3r, v3r, out_ref):
    f32 = jnp.float32
    p1 = p1_ref[...]

    hc = jnp.dot(p1, w1r[...], preferred_element_type=f32)
    m64 = jnp.dot(p1, wmr[...], preferred_element_type=f32)
    h1 = jnp.maximum(hc + bi1r[...], 0.0) * m64[:, :32]

    hp = jnp.pad(h1.astype(jnp.bfloat16), ((40, 40), (0, 0)))
    w2b = w2r[...].astype(jnp.bfloat16)
    acc = jnp.zeros((RS, 64), f32)
    for k in range(9):
        d = (k // 3 - 1) * SX + (k % 3 - 1)
        acc = acc + jnp.dot(hp[40 + d:40 + d + RS, :], w2b[k],
                            preferred_element_type=f32)
    sc2 = g2r[...] * lax.rsqrt(v2r[...] + 1e-5)
    bi2 = b2r[...] - m2r_[...] * sc2
    h2 = jnp.maximum(acc * sc2 + bi2, 0.0) * m64

    h2r = h2.reshape(NB, 30, SX, 64)[:, 1:29, 1:29, :]
    h2s = h2r.reshape(NB, 14, 2, 14, 2, 64)
    acc3 = jnp.zeros((NB * 196, 64), f32)
    for k in range(4):
        dy, dx = divmod(k, 2)
        a = h2s[:, :, dy, :, dx, :].reshape(NB * 196, 64)
        acc3 = acc3 + jnp.dot(a, w3r[k], preferred_element_type=f32)
    sc3 = g3r[...] * lax.rsqrt(v3r[...] + 1e-5)
    bi3 = b3r[...] - m3r[...] * sc3
    h3 = jnp.maximum(acc3 * sc3 + bi3, 0.0) * m2_ref[...]
    out_ref[...] = h3.reshape(NB, 14, 14, 64)


def _run_conv(p1, m2, w1, g1, b1, m1, v1, w2, g2, b2, m2_, v2,
              w3, g3, b3, m3, v3):
    grid = B // NB
    row1 = lambda c: ((1, c), lambda i: (0, 0))
    sc1 = g1 * lax.rsqrt(v1 + 1e-5)
    bi1 = (b1 - m1 * sc1).reshape(1, 32)
    w1s = jnp.zeros((16, 32), jnp.float32)
    w1s = w1s.at[:9, :].set(w1.reshape(9, 32) * sc1[None, :])
    wm = jnp.zeros((16, 64), jnp.float32).at[15, :].set(1.0)
    wspec = [
        pl.BlockSpec((16, 32), lambda i: (0, 0)),
        pl.BlockSpec((16, 64), lambda i: (0, 0)),
        pl.BlockSpec(*row1(32)),
        pl.BlockSpec((9, 32, 64), lambda i: (0, 0, 0)),
        *(pl.BlockSpec(*row1(64)) for _ in range(4)),
        pl.BlockSpec((4, 64, 64), lambda i: (0, 0, 0)),
        *(pl.BlockSpec(*row1(64)) for _ in range(4)),
    ]
    return pl.pallas_call(
        _conv_body,
        grid=(grid,),
        in_specs=[
            pl.BlockSpec((RS, 16), lambda i: (i, 0)),
            pl.BlockSpec((NB * 196, 1), lambda i: (i, 0)),
            *wspec,
        ],
        out_specs=pl.BlockSpec((NB, 14, 14, 64), lambda i: (i, 0, 0, 0)),
        out_shape=jax.ShapeDtypeStruct((B, 14, 14, 64), jnp.float32),
    )(p1, m2, w1s, wm, bi1, w2.reshape(9, 32, 64),
      g2.reshape(1, 64), b2.reshape(1, 64), m2_.reshape(1, 64),
      v2.reshape(1, 64), w3.reshape(4, 64, 64), g3.reshape(1, 64),
      b3.reshape(1, 64), m3.reshape(1, 64), v3.reshape(1, 64))


def _fc_body(a_ref, w1_ref, b1_ref, w2_ref, b2_ref, out_ref):
    f32 = jnp.float32
    z1 = jnp.dot(a_ref[...], w1_ref[...], preferred_element_type=f32)
    z1 = jnp.maximum(z1 + b1_ref[...], 0.0)
    z2 = jnp.dot(z1, w2_ref[...], preferred_element_type=f32) + b2_ref[...]
    col = lax.broadcasted_iota(jnp.int32, (B, 128), 1)
    zm = jnp.where(col < 10, z2, -1e30)
    mx = jnp.max(zm, axis=1, keepdims=True)
    s = jnp.sum(jnp.exp(zm - mx), axis=1, keepdims=True)
    out_ref[...] = z2 - mx - jnp.log(s)


def _run_fc(h3r, fc1_w, fc1_b, fc2_w, fc2_b):
    w1 = fc1_w.reshape(64, 196, 128).transpose(1, 0, 2).reshape(196 * 64, 128)
    w2p = jnp.zeros((128, 128), jnp.float32).at[:, :10].set(fc2_w)
    b2p = jnp.zeros((1, 128), jnp.float32).at[:, :10].set(fc2_b[None, :])
    out = pl.pallas_call(
        _fc_body,
        out_shape=jax.ShapeDtypeStruct((B, 128), jnp.float32),
    )(h3r, w1, fc1_b.reshape(1, 128), w2p, b2p)
    return out[:, :10]


def kernel(features, indices, w1, g1, b1, m1, v1, w2, g2, b2, m2, v2,
           w3, g3, b3, m3, v3, fc1_w, fc1_b, fc2_w, fc2_b):
    flat = (indices[:, 0] * IMG + (indices[:, 1] + 1) * SX
            + indices[:, 2] + 1)
    p1f, m2col = _densify(flat, features[:, 0])
    h3 = _run_conv(p1f.reshape(RTOT, 16), m2col.reshape(B * 196, 1),
                   w1, g1, b1, m1, v1, w2, g2, b2, m2, v2,
                   w3, g3, b3, m3, v3)
    return _run_fc(h3.reshape(B, 196 * 64), fc1_w, fc1_b, fc2_w, fc2_b)


# speedup vs baseline: 1.3703x; 1.3703x over previous
import jax
import jax.numpy as jnp
from jax import lax
from jax.experimental import pallas as pl
from jax.experimental.pallas import tpu as pltpu
from jax.experimental.pallas import tpu_sc as plsc

B = 256
HW = 28
N = 38400
SX = 32
IMG = 30 * SX
RTOT = B * IMG
NW = 32
IPT = B // NW
RCH = IPT * IMG
PCH = 1280
NB = 8
RS = NB * IMG


def _sc_body(flat_hbm, feat_hbm, p1_hbm, m2_hbm, flat_c, feat_c, dense_v,
             taps_v, m2_v, srt_v):
    wid = lax.axis_index("s") * 2 + lax.axis_index("c")
    rbase = wid * RCH
    lanes = lax.iota(jnp.int32, 16)
    zf = jnp.zeros((16,), jnp.float32)
    onef = jnp.ones((16,), jnp.float32)

    def zero_body(v, c):
        dense_v[pl.ds(v * 16, 16)] = zf
        return c

    lax.fori_loop(0, (RCH + 128) // 16, zero_body, 0)

    def zero9_body(v, c):
        taps_v[pl.ds(9 * RCH + v * 16, 16)] = zf
        return c

    lax.fori_loop(0, RCH // 16, zero9_body, 0)

    for j in range(4):
        srt_v[pl.ds(j * 32 + 16, 16)] = jnp.full((16,), -1, jnp.int32)
    big = RCH * 16

    def chunk_body(ci, c):
        pltpu.sync_copy(flat_hbm.at[pl.ds(ci * PCH, PCH)], flat_c)
        pltpu.sync_copy(feat_hbm.at[pl.ds(ci * PCH, PCH)], feat_c)

        def body(i, c2):
            for j in range(4):
                v = i * 4 + j
                rel = flat_c[pl.ds(v * 16, 16)] - rbase
                m = (rel >= 0) & (rel < RCH)
                relc = jnp.where(m, rel, 0)
                f = feat_c[pl.ds(v * 16, 16)]
                key = jnp.where(m, rel * 16 + lanes, big + lanes)
                sk, sv = plsc.sort_key_val(key, f)
                srt_v[pl.ds(j * 32, 16)] = sk
                nxt = srt_v[pl.ds(j * 32 + 1, 16)]
                win = ((sk >> 4) != (nxt >> 4)) & (sk < big)
                dst = jnp.where(sk < big, (sk >> 4) + 64, 0)
                plsc.store_scatter(dense_v, [dst], sv, mask=win)
                plsc.store_scatter(taps_v, [9 * RCH + relc], onef, mask=m)
            return c2

        lax.fori_loop(0, PCH // 64, body, 0)
        return c

    lax.fori_loop(0, N // PCH, chunk_body, 0)

    def exp_body(v, c):
        for k in range(9):
            if k == 4:
                continue
            d = (k // 3 - 1) * SX + (k % 3 - 1)
            taps_v[pl.ds(k * RCH + v * 16, 16)] = dense_v[pl.ds(64 + v * 16 + d, 16)]
        return c

    lax.fori_loop(0, RCH // 16, exp_body, 0)

    def m2_body(v, c):
        l = v * 16 + lanes
        li = l // 196
        pos = l % 196
        y = pos // 14
        x = pos % 14
        r2 = 9 * RCH + li * IMG + (2 * y + 1) * SX + (2 * x + 1)
        g0 = plsc.load_gather(taps_v, [r2])
        g1 = plsc.load_gather(taps_v, [r2 + 1])
        g2 = plsc.load_gather(taps_v, [r2 + SX])
        g3 = plsc.load_gather(taps_v, [r2 + SX + 1])
        m2_v[pl.ds(v * 16, 16)] = jnp.maximum(jnp.maximum(g0, g1),
                                              jnp.maximum(g2, g3))
        return c

    lax.fori_loop(0, IPT * 196 // 16, m2_body, 0)

    for k in range(10):
        if k == 4:
            pltpu.sync_copy(dense_v.at[pl.ds(64, RCH)],
                            p1_hbm.at[pl.ds(k * RTOT + rbase, RCH)])
        else:
            pltpu.sync_copy(taps_v.at[pl.ds(k * RCH, RCH)],
                            p1_hbm.at[pl.ds(k * RTOT + rbase, RCH)])
    pltpu.sync_copy(m2_v, m2_hbm.at[pl.ds(wid * IPT * 196, IPT * 196)])


def _densify(flat, feat):
    mesh = plsc.VectorSubcoreMesh(core_axis_name="c", subcore_axis_name="s")
    return pl.kernel(
        _sc_body,
        mesh=mesh,
        out_type=[jax.ShapeDtypeStruct((10 * RTOT,), jnp.float32),
                  jax.ShapeDtypeStruct((B * 196,), jnp.float32)],
        scratch_types=[pltpu.VMEM((PCH,), jnp.int32),
                       pltpu.VMEM((PCH,), jnp.float32),
                       pltpu.VMEM((RCH + 128,), jnp.float32),
                       pltpu.VMEM((10 * RCH,), jnp.float32),
                       pltpu.VMEM((IPT * 196,), jnp.float32),
                       pltpu.VMEM((128,), jnp.int32)],
        compiler_params=pltpu.CompilerParams(needs_layout_passes=False),
    )(flat, feat)


def _conv_body(p1_ref, m2_ref, w1r, wmr, bi1r, w2r, g2r, b2r,
               m2r_, v2r, w3r, g3r, b3r, m3r, v3r, out_ref):
    f32 = jnp.float32
    p1 = p1_ref[...]

    dn = (((0,), (0,)), ((), ()))
    hc = lax.dot_general(p1, w1r[...], dn, preferred_element_type=f32)
    m64 = lax.dot_general(p1, wmr[...], dn, preferred_element_type=f32)
    h1 = jnp.maximum(hc + bi1r[...], 0.0) * m64[:, :32]

    hp = jnp.pad(h1.astype(jnp.bfloat16), ((40, 40), (0, 0)))
    w2b = w2r[...].astype(jnp.bfloat16)
    acc = jnp.zeros((RS, 64), f32)
    for k in range(9):
        d = (k // 3 - 1) * SX + (k % 3 - 1)
        acc = acc + jnp.dot(hp[40 + d:40 + d + RS, :], w2b[k],
                            preferred_element_type=f32)
    sc2 = g2r[...] * lax.rsqrt(v2r[...] + 1e-5)
    bi2 = b2r[...] - m2r_[...] * sc2
    h2 = jnp.maximum(acc * sc2 + bi2, 0.0) * m64

    h2r = h2.reshape(NB, 30, SX, 64)[:, 1:29, 1:29, :]
    h2s = h2r.reshape(NB, 14, 2, 14, 2, 64)
    acc3 = jnp.zeros((NB * 196, 64), f32)
    for k in range(4):
        dy, dx = divmod(k, 2)
        a = h2s[:, :, dy, :, dx, :].reshape(NB * 196, 64)
        acc3 = acc3 + jnp.dot(a, w3r[k], preferred_element_type=f32)
    sc3 = g3r[...] * lax.rsqrt(v3r[...] + 1e-5)
    bi3 = b3r[...] - m3r[...] * sc3
    h3 = jnp.maximum(acc3 * sc3 + bi3, 0.0) * m2_ref[...]
    out_ref[...] = h3.reshape(NB, 14, 14, 64)


def _run_conv(p1, m2, w1, g1, b1, m1, v1, w2, g2, b2, m2_, v2,
              w3, g3, b3, m3, v3):
    grid = B // NB
    row1 = lambda c: ((1, c), lambda i: (0, 0))
    sc1 = g1 * lax.rsqrt(v1 + 1e-5)
    bi1 = (b1 - m1 * sc1).reshape(1, 32)
    w1s = jnp.zeros((10, 32), jnp.float32)
    w1s = w1s.at[:9, :].set(w1.reshape(9, 32) * sc1[None, :])
    wm = jnp.zeros((10, 64), jnp.float32).at[9, :].set(1.0)
    wspec = [
        pl.BlockSpec((10, 32), lambda i: (0, 0)),
        pl.BlockSpec((10, 64), lambda i: (0, 0)),
        pl.BlockSpec(*row1(32)),
        pl.BlockSpec((9, 32, 64), lambda i: (0, 0, 0)),
        *(pl.BlockSpec(*row1(64)) for _ in range(4)),
        pl.BlockSpec((4, 64, 64), lambda i: (0, 0, 0)),
        *(pl.BlockSpec(*row1(64)) for _ in range(4)),
    ]
    return pl.pallas_call(
        _conv_body,
        grid=(grid,),
        in_specs=[
            pl.BlockSpec((10, RS), lambda i: (0, i)),
            pl.BlockSpec((NB * 196, 1), lambda i: (i, 0)),
            *wspec,
        ],
        out_specs=pl.BlockSpec((NB, 14, 14, 64), lambda i: (i, 0, 0, 0)),
        out_shape=jax.ShapeDtypeStruct((B, 14, 14, 64), jnp.float32),
    )(p1, m2, w1s, wm, bi1, w2.reshape(9, 32, 64),
      g2.reshape(1, 64), b2.reshape(1, 64), m2_.reshape(1, 64),
      v2.reshape(1, 64), w3.reshape(4, 64, 64), g3.reshape(1, 64),
      b3.reshape(1, 64), m3.reshape(1, 64), v3.reshape(1, 64))


def _fc_body(a_ref, w1_ref, b1_ref, w2_ref, b2_ref, out_ref):
    f32 = jnp.float32
    z1 = jnp.dot(a_ref[...], w1_ref[...], preferred_element_type=f32)
    z1 = jnp.maximum(z1 + b1_ref[...], 0.0)
    z2 = jnp.dot(z1, w2_ref[...], preferred_element_type=f32) + b2_ref[...]
    col = lax.broadcasted_iota(jnp.int32, (B, 128), 1)
    zm = jnp.where(col < 10, z2, -1e30)
    mx = jnp.max(zm, axis=1, keepdims=True)
    s = jnp.sum(jnp.exp(zm - mx), axis=1, keepdims=True)
    out_ref[...] = z2 - mx - jnp.log(s)


def _run_fc(h3r, fc1_w, fc1_b, fc2_w, fc2_b):
    w1 = fc1_w.reshape(64, 196, 128).transpose(1, 0, 2).reshape(196 * 64, 128)
    w2p = jnp.zeros((128, 128), jnp.float32).at[:, :10].set(fc2_w)
    b2p = jnp.zeros((1, 128), jnp.float32).at[:, :10].set(fc2_b[None, :])
    out = pl.pallas_call(
        _fc_body,
        out_shape=jax.ShapeDtypeStruct((B, 128), jnp.float32),
    )(h3r, w1, fc1_b.reshape(1, 128), w2p, b2p)
    return out[:, :10]


def kernel(features, indices, w1, g1, b1, m1, v1, w2, g2, b2, m2, v2,
           w3, g3, b3, m3, v3, fc1_w, fc1_b, fc2_w, fc2_b):
    flat = (indices[:, 0] * IMG + (indices[:, 1] + 1) * SX
            + indices[:, 2] + 1)
    p1f, m2col = _densify(flat, features[:, 0])
    h3 = _run_conv(p1f.reshape(10, RTOT), m2col.reshape(B * 196, 1),
                   w1, g1, b1, m1, v1, w2, g2, b2, m2, v2,
                   w3, g3, b3, m3, v3)
    return _run_fc(h3.reshape(B, 196 * 64), fc1_w, fc1_b, fc2_w, fc2_b)


# speedup vs baseline: 1.3727x; 1.0017x over previous
import jax
import jax.numpy as jnp
from jax import lax
from jax.experimental import pallas as pl
from jax.experimental.pallas import tpu as pltpu
from jax.experimental.pallas import tpu_sc as plsc

B = 256
HW = 28
N = 38400
SX = 32
IMG = 30 * SX
RTOT = B * IMG
NW = 32
IPT = B // NW
RCH = IPT * IMG
PCH = 1280
NB = 8
RS = NB * IMG


def _sc_body(flat_hbm, feat_hbm, p1_hbm, m2_hbm, flat_c, feat_c, dense_v,
             taps_v, m2_v, srt0, srt1, srt2, srt3):
    srts = (srt0, srt1, srt2, srt3)
    wid = lax.axis_index("s") * 2 + lax.axis_index("c")
    rbase = wid * RCH
    lanes = lax.iota(jnp.int32, 16)
    zf = jnp.zeros((16,), jnp.float32)
    onef = jnp.ones((16,), jnp.float32)

    def zero_body(v, c):
        dense_v[pl.ds(v * 16, 16)] = zf
        return c

    lax.fori_loop(0, (RCH + 128) // 16, zero_body, 0)

    def zero9_body(v, c):
        taps_v[pl.ds(9 * RCH + v * 16, 16)] = zf
        return c

    lax.fori_loop(0, RCH // 16, zero9_body, 0)

    for j in range(4):
        srts[j][pl.ds(16, 16)] = jnp.full((16,), -1, jnp.int32)
    big = RCH * 16

    def chunk_body(ci, c):
        pltpu.sync_copy(flat_hbm.at[pl.ds(ci * PCH, PCH)], flat_c)
        pltpu.sync_copy(feat_hbm.at[pl.ds(ci * PCH, PCH)], feat_c)

        def body(i, c2):
            for j in range(4):
                v = i * 4 + j
                rel = flat_c[pl.ds(v * 16, 16)] - rbase
                m = (rel >= 0) & (rel < RCH)
                relc = jnp.where(m, rel, 0)
                f = feat_c[pl.ds(v * 16, 16)]
                key = jnp.where(m, rel * 16 + lanes, big + lanes)
                sk, sv = plsc.sort_key_val(key, f)
                srts[j][pl.ds(0, 16)] = sk
                nxt = srts[j][pl.ds(1, 16)]
                win = ((sk >> 4) != (nxt >> 4)) & (sk < big)
                dst = jnp.where(sk < big, (sk >> 4) + 64, 0)
                plsc.store_scatter(dense_v, [dst], sv, mask=win)
                plsc.store_scatter(taps_v, [9 * RCH + relc], onef, mask=m)
            return c2

        lax.fori_loop(0, PCH // 64, body, 0)
        return c

    lax.fori_loop(0, N // PCH, chunk_body, 0)

    def exp_body(v, c):
        for k in range(9):
            if k == 4:
                continue
            d = (k // 3 - 1) * SX + (k % 3 - 1)
            taps_v[pl.ds(k * RCH + v * 16, 16)] = dense_v[pl.ds(64 + v * 16 + d, 16)]
        return c

    lax.fori_loop(0, RCH // 16, exp_body, 0)

    def m2_body(v, c):
        l = v * 16 + lanes
        li = l // 196
        pos = l % 196
        y = pos // 14
        x = pos % 14
        r2 = 9 * RCH + li * IMG + (2 * y + 1) * SX + (2 * x + 1)
        g0 = plsc.load_gather(taps_v, [r2])
        g1 = plsc.load_gather(taps_v, [r2 + 1])
        g2 = plsc.load_gather(taps_v, [r2 + SX])
        g3 = plsc.load_gather(taps_v, [r2 + SX + 1])
        m2_v[pl.ds(v * 16, 16)] = jnp.maximum(jnp.maximum(g0, g1),
                                              jnp.maximum(g2, g3))
        return c

    lax.fori_loop(0, IPT * 196 // 16, m2_body, 0)

    for k in range(10):
        if k == 4:
            pltpu.sync_copy(dense_v.at[pl.ds(64, RCH)],
                            p1_hbm.at[pl.ds(k * RTOT + rbase, RCH)])
        else:
            pltpu.sync_copy(taps_v.at[pl.ds(k * RCH, RCH)],
                            p1_hbm.at[pl.ds(k * RTOT + rbase, RCH)])
    pltpu.sync_copy(m2_v, m2_hbm.at[pl.ds(wid * IPT * 196, IPT * 196)])


def _densify(flat, feat):
    mesh = plsc.VectorSubcoreMesh(core_axis_name="c", subcore_axis_name="s")
    return pl.kernel(
        _sc_body,
        mesh=mesh,
        out_type=[jax.ShapeDtypeStruct((10 * RTOT,), jnp.float32),
                  jax.ShapeDtypeStruct((B * 196,), jnp.float32)],
        scratch_types=[pltpu.VMEM((PCH,), jnp.int32),
                       pltpu.VMEM((PCH,), jnp.float32),
                       pltpu.VMEM((RCH + 128,), jnp.float32),
                       pltpu.VMEM((10 * RCH,), jnp.float32),
                       pltpu.VMEM((IPT * 196,), jnp.float32),
                       pltpu.VMEM((32,), jnp.int32),
                       pltpu.VMEM((32,), jnp.int32),
                       pltpu.VMEM((32,), jnp.int32),
                       pltpu.VMEM((32,), jnp.int32)],
        compiler_params=pltpu.CompilerParams(needs_layout_passes=False),
    )(flat, feat)


def _conv_body(p1_ref, m2_ref, w1r, wmr, bi1r, w2r, g2r, b2r,
               m2r_, v2r, w3r, g3r, b3r, m3r, v3r, out_ref):
    f32 = jnp.float32
    p1 = p1_ref[...]

    dn = (((0,), (0,)), ((), ()))
    hc = lax.dot_general(p1, w1r[...], dn, preferred_element_type=f32)
    m64 = lax.dot_general(p1, wmr[...], dn, preferred_element_type=f32)
    h1 = jnp.maximum(hc + bi1r[...], 0.0) * m64[:, :32]

    hp = jnp.pad(h1, ((40, 40), (0, 0)))
    w2b = w2r[...]
    acc = jnp.zeros((RS, 64), f32)
    for k in range(9):
        d = (k // 3 - 1) * SX + (k % 3 - 1)
        acc = acc + jnp.dot(hp[40 + d:40 + d + RS, :], w2b[k],
                            preferred_element_type=f32)
    sc2 = g2r[...] * lax.rsqrt(v2r[...] + 1e-5)
    bi2 = b2r[...] - m2r_[...] * sc2
    h2 = jnp.maximum(acc * sc2 + bi2, 0.0) * m64

    h2r = h2.reshape(NB, 30, SX, 64)[:, 1:29, 1:29, :]
    h2s = h2r.reshape(NB, 14, 2, 14, 2, 64)
    acc3 = jnp.zeros((NB * 196, 64), f32)
    for k in range(4):
        dy, dx = divmod(k, 2)
        a = h2s[:, :, dy, :, dx, :].reshape(NB * 196, 64)
        acc3 = acc3 + jnp.dot(a, w3r[k], preferred_element_type=f32)
    sc3 = g3r[...] * lax.rsqrt(v3r[...] + 1e-5)
    bi3 = b3r[...] - m3r[...] * sc3
    h3 = jnp.maximum(acc3 * sc3 + bi3, 0.0) * m2_ref[...]
    out_ref[...] = h3.reshape(NB, 14, 14, 64)


def _run_conv(p1, m2, w1, g1, b1, m1, v1, w2, g2, b2, m2_, v2,
              w3, g3, b3, m3, v3):
    grid = B // NB
    row1 = lambda c: ((1, c), lambda i: (0, 0))
    sc1 = g1 * lax.rsqrt(v1 + 1e-5)
    bi1 = (b1 - m1 * sc1).reshape(1, 32)
    w1s = jnp.zeros((10, 32), jnp.float32)
    w1s = w1s.at[:9, :].set(w1.reshape(9, 32) * sc1[None, :])
    wm = jnp.zeros((10, 64), jnp.float32).at[9, :].set(1.0)
    wspec = [
        pl.BlockSpec((10, 32), lambda i: (0, 0)),
        pl.BlockSpec((10, 64), lambda i: (0, 0)),
        pl.BlockSpec(*row1(32)),
        pl.BlockSpec((9, 32, 64), lambda i: (0, 0, 0)),
        *(pl.BlockSpec(*row1(64)) for _ in range(4)),
        pl.BlockSpec((4, 64, 64), lambda i: (0, 0, 0)),
        *(pl.BlockSpec(*row1(64)) for _ in range(4)),
    ]
    return pl.pallas_call(
        _conv_body,
        grid=(grid,),
        in_specs=[
            pl.BlockSpec((10, RS), lambda i: (0, i)),
            pl.BlockSpec((NB * 196, 1), lambda i: (i, 0)),
            *wspec,
        ],
        out_specs=pl.BlockSpec((NB, 14, 14, 64), lambda i: (i, 0, 0, 0)),
        out_shape=jax.ShapeDtypeStruct((B, 14, 14, 64), jnp.float32),
    )(p1, m2, w1s, wm, bi1, w2.reshape(9, 32, 64),
      g2.reshape(1, 64), b2.reshape(1, 64), m2_.reshape(1, 64),
      v2.reshape(1, 64), w3.reshape(4, 64, 64), g3.reshape(1, 64),
      b3.reshape(1, 64), m3.reshape(1, 64), v3.reshape(1, 64))


def _fc_body(a_ref, w1_ref, b1_ref, w2_ref, b2_ref, out_ref):
    f32 = jnp.float32
    z1 = jnp.dot(a_ref[...], w1_ref[...], preferred_element_type=f32)
    z1 = jnp.maximum(z1 + b1_ref[...], 0.0)
    z2 = jnp.dot(z1, w2_ref[...], preferred_element_type=f32) + b2_ref[...]
    col = lax.broadcasted_iota(jnp.int32, (B, 128), 1)
    zm = jnp.where(col < 10, z2, -1e30)
    mx = jnp.max(zm, axis=1, keepdims=True)
    s = jnp.sum(jnp.exp(zm - mx), axis=1, keepdims=True)
    out_ref[...] = z2 - mx - jnp.log(s)


def _run_fc(h3r, fc1_w, fc1_b, fc2_w, fc2_b):
    w1 = fc1_w.reshape(64, 196, 128).transpose(1, 0, 2).reshape(196 * 64, 128)
    w2p = jnp.zeros((128, 128), jnp.float32).at[:, :10].set(fc2_w)
    b2p = jnp.zeros((1, 128), jnp.float32).at[:, :10].set(fc2_b[None, :])
    out = pl.pallas_call(
        _fc_body,
        out_shape=jax.ShapeDtypeStruct((B, 128), jnp.float32),
    )(h3r, w1, fc1_b.reshape(1, 128), w2p, b2p)
    return out[:, :10]


def kernel(features, indices, w1, g1, b1, m1, v1, w2, g2, b2, m2, v2,
           w3, g3, b3, m3, v3, fc1_w, fc1_b, fc2_w, fc2_b):
    flat = (indices[:, 0] * IMG + (indices[:, 1] + 1) * SX
            + indices[:, 2] + 1)
    p1f, m2col = _densify(flat, features[:, 0])
    h3 = _run_conv(p1f.reshape(10, RTOT), m2col.reshape(B * 196, 1),
                   w1, g1, b1, m1, v1, w2, g2, b2, m2, v2,
                   w3, g3, b3, m3, v3)
    return _run_fc(h3.reshape(B, 196 * 64), fc1_w, fc1_b, fc2_w, fc2_b)
